# baseline (speedup 1.0000x reference)
import functools

import jax
import jax.numpy as jnp
from jax import lax
from jax.experimental import pallas as pl
from jax.experimental.pallas import tpu as pltpu
from jax.experimental.pallas import tpu_sc as plsc

B, T = 16384, 200
NC, NS, L = 2, 16, 16
NW = NC * NS
CPW = B // NW
SWAP_GROUPS = CPW // L

_OUT_T = jax.ShapeDtypeStruct((T, B), jnp.float32)


@functools.partial(
    pl.kernel,
    out_type=(_OUT_T, _OUT_T),
    mesh=plsc.VectorSubcoreMesh(core_axis_name="c", subcore_axis_name="s"),
    scratch_types=[pltpu.VMEM((T, CPW), jnp.float32)],
    compiler_params=pltpu.CompilerParams(
        use_tc_tiling_on_sc=True, needs_layout_passes=False
    ),
)
def _swap_sc(xt_hbm, o1_hbm, o3_hbm, buf):
    wid = lax.axis_index("s") * NC + lax.axis_index("c")
    cols = pl.ds(wid * CPW, CPW)
    pltpu.sync_copy(xt_hbm.at[:, cols], buf)
    lanes = lax.iota(jnp.int32, L)
    row0 = jnp.zeros((L,), jnp.int32)
    row1 = row0 + 1
    for g in range(SWAP_GROUPS):
        c = lanes + (g * L)
        v0 = plsc.load_gather(buf, [row0, c])
        v1 = plsc.load_gather(buf, [row1, c])
        plsc.store_scatter(buf, [row0, c], v1)
        plsc.store_scatter(buf, [row1, c], v0)
    pltpu.sync_copy(buf, o1_hbm.at[:, cols])
    pltpu.sync_copy(buf, o3_hbm.at[:, cols])


_BN = 4096


def _fanout_body(xt_ref, o0_ref, o2_ref, o4_ref):
    v = xt_ref[...]
    o0_ref[...] = v
    o2_ref[...] = v
    o4_ref[...] = v


_fanout_tc = pl.pallas_call(
    _fanout_body,
    grid=(B // _BN,),
    in_specs=[pl.BlockSpec((T, _BN), lambda i: (0, i))],
    out_specs=[pl.BlockSpec((T, _BN), lambda i: (0, i)) for _ in range(3)],
    out_shape=[_OUT_T for _ in range(3)],
)


def kernel(x):
    xt = x.T
    o1, o3 = _swap_sc(xt)
    o0, o2, o4 = _fanout_tc(xt)
    return (o0.T, o1.T, o2.T, o3.T, o4.T)


# speedup vs baseline: 1.0029x; 1.0029x over previous
import functools

import jax
import jax.numpy as jnp
from jax import lax
from jax.experimental import pallas as pl
from jax.experimental.pallas import tpu as pltpu
from jax.experimental.pallas import tpu_sc as plsc

B, T = 16384, 200
NC, NS, L = 2, 16, 16
NW = NC * NS
CPW = B // NW
SWAP_GROUPS = CPW // L

_OUT_T = jax.ShapeDtypeStruct((T, B), jnp.float32)


@functools.partial(
    pl.kernel,
    out_type=(_OUT_T, _OUT_T),
    mesh=plsc.VectorSubcoreMesh(core_axis_name="c", subcore_axis_name="s"),
    scratch_types=[pltpu.VMEM((T, CPW), jnp.float32)],
    compiler_params=pltpu.CompilerParams(
        use_tc_tiling_on_sc=True, needs_layout_passes=False, skip_device_barrier=True
    ),
)
def _swap_sc(xt_hbm, o1_hbm, o3_hbm, buf):
    wid = lax.axis_index("s") * NC + lax.axis_index("c")
    cols = pl.ds(wid * CPW, CPW)
    pltpu.sync_copy(xt_hbm.at[:, cols], buf)
    lanes = lax.iota(jnp.int32, L)
    row0 = jnp.zeros((L,), jnp.int32)
    row1 = row0 + 1
    for g in range(SWAP_GROUPS):
        c = lanes + (g * L)
        v0 = plsc.load_gather(buf, [row0, c])
        v1 = plsc.load_gather(buf, [row1, c])
        plsc.store_scatter(buf, [row0, c], v1)
        plsc.store_scatter(buf, [row1, c], v0)
    pltpu.sync_copy(buf, o1_hbm.at[:, cols])
    pltpu.sync_copy(buf, o3_hbm.at[:, cols])


_BN = 2048


def _fanout_body(xt_ref, o0_ref, o2_ref, o4_ref):
    v = xt_ref[...]
    o0_ref[...] = v
    o2_ref[...] = v
    o4_ref[...] = v


_fanout_tc = pl.pallas_call(
    _fanout_body,
    grid=(B // _BN,),
    in_specs=[pl.BlockSpec((T, _BN), lambda i: (0, i))],
    out_specs=[pl.BlockSpec((T, _BN), lambda i: (0, i)) for _ in range(3)],
    out_shape=[_OUT_T for _ in range(3)],
)


def kernel(x):
    xt = x.T
    o1, o3 = _swap_sc(xt)
    o0, o2, o4 = _fanout_tc(xt)
    return (o0.T, o1.T, o2.T, o3.T, o4.T)
